# baseline (speedup 1.0000x reference)
import functools

import jax
import jax.numpy as jnp
from jax import lax
from jax.experimental import pallas as pl
from jax.experimental.pallas import tpu as pltpu
from jax.experimental.pallas import tpu_sc as plsc

D_MODEL = 128
MAX_LEN = 1000

_NUM_CORES = 2
_NUM_SUBCORES = 16
_NUM_WORKERS = _NUM_CORES * _NUM_SUBCORES
_LANES = 16

_CHUNK = 128
_PAIRS = D_MODEL // 2
_HALF_GROUPS = _PAIRS // _LANES


def _sc_body(x_hbm, ts_hbm, pe_hbm, out_hbm, idx_all, xbuf, perows,
             sem_in, sem_wb, *, chunks_per_worker):
    wid = lax.axis_index("s") * _NUM_CORES + lax.axis_index("c")
    idx_row0 = wid * chunks_per_worker
    row0 = idx_row0 * _CHUNK

    pltpu.sync_copy(ts_hbm.at[pl.ds(idx_row0, chunks_per_worker)], idx_all)

    def clamp_row(c, carry):
        for j in range(_CHUNK // _LANES):
            s = pl.ds(j * _LANES, _LANES)
            idx_all[c, s] = jnp.minimum(jnp.maximum(idx_all[c, s], 0),
                                        MAX_LEN - 1)
        return carry

    lax.fori_loop(0, chunks_per_worker, clamp_row, 0, unroll=False)

    def start_fetch(c, slot):
        pltpu.async_copy(pe_hbm.at[idx_all.at[c]], perows.at[slot],
                         sem_in.at[slot])
        pltpu.async_copy(x_hbm.at[pl.ds(row0 + c * _CHUNK, _CHUNK)],
                         xbuf.at[slot], sem_in.at[slot])

    def wait_fetch(c, slot):
        pltpu.make_async_copy(pe_hbm.at[idx_all.at[c]], perows.at[slot],
                              sem_in.at[slot]).wait()
        pltpu.make_async_copy(x_hbm.at[pl.ds(row0 + c * _CHUNK, _CHUNK)],
                              xbuf.at[slot], sem_in.at[slot]).wait()

    def wait_wb(c, slot):
        pltpu.make_async_copy(xbuf.at[slot],
                              out_hbm.at[pl.ds(row0 + c * _CHUNK, _CHUNK)],
                              sem_wb.at[slot]).wait()

    start_fetch(0, 0)

    def pair_body(g, carry):
        for slot in range(2):
            c = 2 * g + slot
            other = 1 - slot

            @pl.when(c + 1 < chunks_per_worker)
            def _():
                @pl.when(c >= 1)
                def _():
                    wait_wb(c - 1, other)
                start_fetch(c + 1, other)

            wait_fetch(c, slot)

            def add_row(r, carry2, slot=slot):
                for j in range(_HALF_GROUPS):
                    w = perows[slot, r, pl.ds(j * _LANES, _LANES)]
                    lo = lax.bitcast_convert_type(
                        lax.shift_left(w, 16), jnp.float32)
                    hi = lax.bitcast_convert_type(
                        lax.bitwise_and(w, jnp.int32(-65536)), jnp.float32)
                    sl = pl.ds(j * _LANES, _LANES)
                    sh = pl.ds(_PAIRS + j * _LANES, _LANES)
                    xbuf[slot, r, sl] = xbuf[slot, r, sl] + lo
                    xbuf[slot, r, sh] = xbuf[slot, r, sh] + hi
                return carry2

            lax.fori_loop(0, _CHUNK, add_row, 0, unroll=False)

            pltpu.async_copy(xbuf.at[slot],
                             out_hbm.at[pl.ds(row0 + c * _CHUNK, _CHUNK)],
                             sem_wb.at[slot])
        return carry

    lax.fori_loop(0, chunks_per_worker // 2, pair_body, 0, unroll=False)
    wait_wb(chunks_per_worker - 2, 0)
    wait_wb(chunks_per_worker - 1, 1)


def kernel(x, timestamps, pe):
    batch, seq, d = x.shape
    n = batch * seq
    assert d == D_MODEL and n % (_NUM_WORKERS * _CHUNK) == 0
    chunks_per_worker = n // (_NUM_WORKERS * _CHUNK)
    assert chunks_per_worker >= 2 and chunks_per_worker % 2 == 0

    x2 = x.reshape(n, d)
    ts2 = timestamps.astype(jnp.int32).reshape(n // _CHUNK, _CHUNK)

    pe_bits = lax.bitcast_convert_type(
        pe.astype(jnp.bfloat16), jnp.uint16).astype(jnp.uint32)
    pe_packed = lax.bitcast_convert_type(
        (pe_bits[:, _PAIRS:] << 16) | pe_bits[:, :_PAIRS], jnp.int32)

    mesh = plsc.VectorSubcoreMesh(core_axis_name="c", subcore_axis_name="s")
    body = functools.partial(_sc_body, chunks_per_worker=chunks_per_worker)
    out = pl.kernel(
        body,
        out_type=jax.ShapeDtypeStruct((n, d), jnp.float32),
        mesh=mesh,
        compiler_params=pltpu.CompilerParams(use_tc_tiling_on_sc=False),
        scratch_types=[
            pltpu.VMEM((chunks_per_worker, _CHUNK), jnp.int32),
            pltpu.VMEM((2, _CHUNK, D_MODEL), jnp.float32),
            pltpu.VMEM((2, _CHUNK, _PAIRS), jnp.int32),
            pltpu.SemaphoreType.DMA((2,)),
            pltpu.SemaphoreType.DMA((2,)),
        ],
    )(x2, ts2, pe_packed)
    return out.reshape(batch, seq, d)


# speedup vs baseline: 1.5470x; 1.5470x over previous
import functools

import jax
import jax.numpy as jnp
from jax import lax
from jax.experimental import pallas as pl
from jax.experimental.pallas import tpu as pltpu
from jax.experimental.pallas import tpu_sc as plsc

D_MODEL = 128
MAX_LEN = 1000

_NUM_CORES = 2
_NUM_SUBCORES = 16
_NUM_WORKERS = _NUM_CORES * _NUM_SUBCORES
_LANES = 16
_REPLICAS = 8

_CHUNK = 128


def _sc_body(x_hbm, ts_hbm, pe_hbm, out_hbm, idx_all, xbuf, perows,
             sem_in, sem_wb, *, chunks_per_worker):
    wid = lax.axis_index("s") * _NUM_CORES + lax.axis_index("c")
    idx_row0 = wid * chunks_per_worker
    row0 = idx_row0 * _CHUNK
    pe_copy = pe_hbm.at[lax.rem(wid, _REPLICAS)]

    pltpu.sync_copy(ts_hbm.at[pl.ds(idx_row0, chunks_per_worker)], idx_all)

    def clamp_row(c, carry):
        for j in range(_CHUNK // _LANES):
            s = pl.ds(j * _LANES, _LANES)
            idx_all[c, s] = jnp.minimum(jnp.maximum(idx_all[c, s], 0),
                                        MAX_LEN - 1)
        return carry

    lax.fori_loop(0, chunks_per_worker, clamp_row, 0, unroll=False)

    def start_fetch(c, slot):
        pltpu.async_copy(pe_copy.at[idx_all.at[c]], perows.at[slot],
                         sem_in.at[slot])
        pltpu.async_copy(x_hbm.at[pl.ds(row0 + c * _CHUNK, _CHUNK)],
                         xbuf.at[slot], sem_in.at[slot])

    def wait_fetch(c, slot):
        pltpu.make_async_copy(pe_copy.at[idx_all.at[c]], perows.at[slot],
                              sem_in.at[slot]).wait()
        pltpu.make_async_copy(x_hbm.at[pl.ds(row0 + c * _CHUNK, _CHUNK)],
                              xbuf.at[slot], sem_in.at[slot]).wait()

    def wait_wb(c, slot):
        pltpu.make_async_copy(xbuf.at[slot],
                              out_hbm.at[pl.ds(row0 + c * _CHUNK, _CHUNK)],
                              sem_wb.at[slot]).wait()

    start_fetch(0, 0)

    def pair_body(g, carry):
        for slot in range(2):
            c = 2 * g + slot
            other = 1 - slot

            @pl.when(c + 1 < chunks_per_worker)
            def _():
                @pl.when(c >= 1)
                def _():
                    wait_wb(c - 1, other)
                start_fetch(c + 1, other)

            wait_fetch(c, slot)

            def add_row(r, carry2, slot=slot):
                for j in range(D_MODEL // _LANES):
                    s = pl.ds(j * _LANES, _LANES)
                    xbuf[slot, r, s] = xbuf[slot, r, s] + perows[slot, r, s]
                return carry2

            lax.fori_loop(0, _CHUNK, add_row, 0, unroll=False)

            pltpu.async_copy(xbuf.at[slot],
                             out_hbm.at[pl.ds(row0 + c * _CHUNK, _CHUNK)],
                             sem_wb.at[slot])
        return carry

    lax.fori_loop(0, chunks_per_worker // 2, pair_body, 0, unroll=False)
    wait_wb(chunks_per_worker - 2, 0)
    wait_wb(chunks_per_worker - 1, 1)


def kernel(x, timestamps, pe):
    batch, seq, d = x.shape
    n = batch * seq
    assert d == D_MODEL and n % (_NUM_WORKERS * _CHUNK) == 0
    chunks_per_worker = n // (_NUM_WORKERS * _CHUNK)
    assert chunks_per_worker >= 2 and chunks_per_worker % 2 == 0

    x2 = x.reshape(n, d)
    ts2 = timestamps.astype(jnp.int32).reshape(n // _CHUNK, _CHUNK)
    pe_rep = jnp.broadcast_to(pe, (_REPLICAS,) + pe.shape)

    mesh = plsc.VectorSubcoreMesh(core_axis_name="c", subcore_axis_name="s")
    body = functools.partial(_sc_body, chunks_per_worker=chunks_per_worker)
    out = pl.kernel(
        body,
        out_type=jax.ShapeDtypeStruct((n, d), jnp.float32),
        mesh=mesh,
        scratch_types=[
            pltpu.VMEM((chunks_per_worker, _CHUNK), jnp.int32),
            pltpu.VMEM((2, _CHUNK, D_MODEL), jnp.float32),
            pltpu.VMEM((2, _CHUNK, D_MODEL), jnp.float32),
            pltpu.SemaphoreType.DMA((2,)),
            pltpu.SemaphoreType.DMA((2,)),
        ],
    )(x2, ts2, pe_rep)
    return out.reshape(batch, seq, d)
